# baseline (speedup 1.0000x reference)
import jax
import jax.numpy as jnp
from jax import lax
from jax.experimental import pallas as pl
from jax.experimental.pallas import tpu as pltpu
from jax.experimental.pallas import tpu_sc as plsc

N = 10000
E = 320000
D_IN = 128
D_HID = 16
D_OUT = 128

NP = 10240
NF = NP // 8
JUNK = NP
ACC_ROWS = NP + 16
TILES = 32
CH = 128
NCHT = 2560
NCH0 = 70
NCH1 = 90
DCH0 = 63
DCH1 = 97
K = 5
NG0 = NCH0 // K
NG1 = NCH1 // K

_mesh = plsc.VectorSubcoreMesh(core_axis_name="c", subcore_axis_name="s")
_sc_params = pltpu.CompilerParams(use_tc_tiling_on_sc=False)


def _pack_to_flat(ab, fb):
    @pl.loop(0, 80)
    def _(i):
        for p in range(8):
            fb[i, p * D_HID:(p + 1) * D_HID] = ab[i * 8 + p, :]


def _unpack_from_flat(fb, ab):
    @pl.loop(0, 80)
    def _(i):
        for p in range(8):
            ab[i * 8 + p, :] = fb[i, p * D_HID:(p + 1) * D_HID]


def _deg_body(idx_hbm, out0_hbm, out1_hbm, dst_v, ones_v, zbuf, tbuf, acc,
              sem):
    c = lax.axis_index("c")
    s = lax.axis_index("s")

    @pl.loop(0, 641)
    def _(i):
        zbuf[i, :] = jnp.zeros((16,), jnp.float32)

    @pl.loop(0, CH)
    def _(i):
        ones_v[i, :] = jnp.ones((16,), jnp.float32)

    pltpu.sync_copy(zbuf, acc.at[pl.ds(s * 641, 641)])

    def scatter_ones(nch, base):
        pltpu.sync_copy(idx_hbm.at[1, pl.ds(base, nch)],
                        dst_v.at[pl.ds(0, nch)])
        plsc.subcore_barrier()

        @pl.loop(0, nch)
        def _(j):
            pltpu.async_copy(ones_v, acc.at[dst_v.at[j]], sem, add=True)

        @pl.loop(0, nch)
        def _(j):
            pltpu.make_async_copy(ones_v, acc.at[dst_v.at[0]], sem).wait()

    @pl.when(c == 0)
    def _():
        scatter_ones(DCH0, s * DCH0)

    @pl.when(c == 1)
    def _():
        scatter_ones(DCH1, 16 * DCH0 + s * DCH1)

    plsc.subcore_barrier()
    pltpu.sync_copy(acc.at[pl.ds(s * 640, 640)], zbuf.at[pl.ds(0, 640)])
    _pack_to_flat(zbuf, tbuf)

    @pl.when(c == 0)
    def _():
        pltpu.sync_copy(tbuf, out0_hbm.at[pl.ds(s * 80, 80)])

    @pl.when(c == 1)
    def _():
        pltpu.sync_copy(tbuf, out1_hbm.at[pl.ds(s * 80, 80)])


_deg = pl.kernel(
    _deg_body,
    out_type=(jax.ShapeDtypeStruct((NF, 128), jnp.float32),
              jax.ShapeDtypeStruct((NF, 128), jnp.float32)),
    mesh=_mesh,
    scratch_types=[
        pltpu.VMEM((DCH1, CH), jnp.int32),
        pltpu.VMEM((CH, D_HID), jnp.float32),
        pltpu.VMEM((641, D_HID), jnp.float32),
        pltpu.VMEM((80, 128), jnp.float32),
        pltpu.VMEM_SHARED((ACC_ROWS, D_HID), jnp.float32),
        pltpu.SemaphoreType.DMA,
    ],
    compiler_params=_sc_params,
)


def _prop_body(gf_hbm, idx_hbm, out0_hbm, out1_hbm, src_v, dst_v, rows, zbuf,
               tbuf, abuf, gsh, acc, gsem0, gsem1, ssem0, ssem1):
    c = lax.axis_index("c")
    s = lax.axis_index("s")

    @pl.loop(0, 641)
    def _(i):
        zbuf[i, :] = jnp.zeros((16,), jnp.float32)

    pltpu.sync_copy(zbuf, acc.at[pl.ds(s * 641, 641)])
    pltpu.sync_copy(gf_hbm.at[pl.ds(s * 80, 80)], tbuf)
    _unpack_from_flat(tbuf, abuf)
    pltpu.sync_copy(abuf, gsh.at[pl.ds(s * 640, 640)])

    def fire_gathers(gbase, st, gsem):
        for k in range(K):
            pltpu.async_copy(gsh.at[src_v.at[gbase + k]], rows.at[st, k],
                             gsem)

    def wait_gathers(st, gsem):
        for k in range(K):
            pltpu.make_async_copy(gsh.at[src_v.at[0]], rows.at[st, k],
                                  gsem).wait()

    def fire_scatters(gbase, st, ssem):
        for k in range(K):
            pltpu.async_copy(rows.at[st, k], acc.at[dst_v.at[gbase + k]],
                             ssem, add=True)

    def wait_scatters(st, ssem):
        for k in range(K):
            pltpu.make_async_copy(rows.at[st, k], acc.at[dst_v.at[0]],
                                  ssem).wait()

    def pipeline(nch, ng, base):
        pltpu.sync_copy(idx_hbm.at[0, pl.ds(base, nch)],
                        src_v.at[pl.ds(0, nch)])
        pltpu.sync_copy(idx_hbm.at[1, pl.ds(base, nch)],
                        dst_v.at[pl.ds(0, nch)])
        plsc.subcore_barrier()

        fire_gathers(0, 0, gsem0)
        fire_gathers(K, 1, gsem1)
        wait_gathers(0, gsem0)
        fire_scatters(0, 0, ssem0)

        @pl.loop(0, (ng - 2) // 2)
        def _(i):
            ga = 1 + 2 * i
            wait_scatters(0, ssem0)
            fire_gathers((ga + 1) * K, 0, gsem0)
            wait_gathers(1, gsem1)
            fire_scatters(ga * K, 1, ssem1)
            gb = ga + 1
            wait_scatters(1, ssem1)
            fire_gathers((gb + 1) * K, 1, gsem1)
            wait_gathers(0, gsem0)
            fire_scatters(gb * K, 0, ssem0)

        wait_scatters(0, ssem0)
        wait_gathers(1, gsem1)
        fire_scatters((ng - 1) * K, 1, ssem1)
        wait_scatters(1, ssem1)

    @pl.when(c == 0)
    def _():
        pipeline(NCH0, NG0, s * NCH0)

    @pl.when(c == 1)
    def _():
        pipeline(NCH1, NG1, 16 * NCH0 + s * NCH1)

    plsc.subcore_barrier()
    pltpu.sync_copy(acc.at[pl.ds(s * 640, 640)], abuf)
    _pack_to_flat(abuf, tbuf)

    @pl.when(c == 0)
    def _():
        pltpu.sync_copy(tbuf, out0_hbm.at[pl.ds(s * 80, 80)])

    @pl.when(c == 1)
    def _():
        pltpu.sync_copy(tbuf, out1_hbm.at[pl.ds(s * 80, 80)])


_prop = pl.kernel(
    _prop_body,
    out_type=(jax.ShapeDtypeStruct((NF, 128), jnp.float32),
              jax.ShapeDtypeStruct((NF, 128), jnp.float32)),
    mesh=_mesh,
    scratch_types=[
        pltpu.VMEM((NCH1, CH), jnp.int32),
        pltpu.VMEM((NCH1, CH), jnp.int32),
        pltpu.VMEM((2, K, CH, D_HID), jnp.float32),
        pltpu.VMEM((641, D_HID), jnp.float32),
        pltpu.VMEM((80, 128), jnp.float32),
        pltpu.VMEM((640, D_HID), jnp.float32),
        pltpu.VMEM_SHARED((NP, D_HID), jnp.float32),
        pltpu.VMEM_SHARED((ACC_ROWS, D_HID), jnp.float32),
        pltpu.SemaphoreType.DMA,
        pltpu.SemaphoreType.DMA,
        pltpu.SemaphoreType.DMA,
        pltpu.SemaphoreType.DMA,
    ],
    compiler_params=_sc_params,
)


def _scale_body(da_ref, db_ref, x_ref, w_ref, g_ref, dis_ref):
    deg = da_ref[...] + db_ref[...] + 1.0
    dis = lax.rsqrt(deg)
    dis_ref[...] = dis
    for k in range(8):
        hk = jnp.dot(x_ref[k], w_ref[...], preferred_element_type=jnp.float32)
        g_ref[:, k * D_HID:(k + 1) * D_HID] = hk
    g_ref[...] = dis * g_ref[...]


def _mid_body(aa_ref, ab_ref, dis_ref, g_ref, b_ref, g2_ref):
    dis = dis_ref[...]
    z = jnp.maximum(dis * (aa_ref[...] + ab_ref[...] + g_ref[...])
                    + b_ref[...], 0.0)
    g2_ref[...] = dis * z


def _out_body(aa_ref, ab_ref, dis_ref, g2_ref, w_ref, b_ref, o_ref):
    a2 = dis_ref[...] * (aa_ref[...] + ab_ref[...] + g2_ref[...])
    for k in range(8):
        v = jnp.dot(a2[:, k * D_HID:(k + 1) * D_HID], w_ref[...],
                    preferred_element_type=jnp.float32) + b_ref[...]
        if (k + 1) * NF <= N:
            o_ref[pl.ds(k * NF, NF)] = v
        else:
            o_ref[pl.ds(k * NF, N - k * NF)] = v[:N - k * NF]


_f32 = jnp.float32
_S = jax.ShapeDtypeStruct

_scale = pl.pallas_call(
    _scale_body,
    out_shape=(_S((NF, 128), _f32), _S((NF, 128), _f32)))
_mid = pl.pallas_call(_mid_body, out_shape=_S((NF, 128), _f32))
_out = pl.pallas_call(_out_body, out_shape=_S((N, D_OUT), _f32))


def kernel(x, edge_index, batch, W1, b1, W2, b2):
    q = jnp.floor((edge_index.astype(jnp.float32) + 0.5)
                  * (1.0 / NF)).astype(jnp.int32)
    ei = (edge_index - q * NF) * 8 + q
    pad_e = NCHT * CH - E
    pad_cols = jnp.concatenate(
        [jnp.zeros((1, pad_e), jnp.int32),
         jnp.full((1, pad_e), JUNK, jnp.int32)])
    idx_r = jnp.concatenate([ei, pad_cols], axis=1).reshape(2, NCHT, CH)
    x_r = jnp.pad(x, ((0, NP - N), (0, 0))).reshape(8, NF, D_IN)
    b1f = jnp.tile(b1, 8).reshape(1, 128)

    dega, degb = _deg(idx_r)
    g1, dis = _scale(dega, degb, x_r, W1)
    a1a, a1b = _prop(g1, idx_r)
    g2 = _mid(a1a, a1b, dis, g1, b1f)
    a2a, a2b = _prop(g2, idx_r)
    return _out(a2a, a2b, dis, g2, W2, b2.reshape(1, D_OUT))


# speedup vs baseline: 1.0278x; 1.0278x over previous
import jax
import jax.numpy as jnp
from jax import lax
from jax.experimental import pallas as pl
from jax.experimental.pallas import tpu as pltpu
from jax.experimental.pallas import tpu_sc as plsc

N = 10000
E = 320000
D_IN = 128
D_HID = 16
D_OUT = 128

NP = 10240
NF = NP // 8
JUNK = NP
ACC_ROWS = NP + 16
TILES = 32
CH = 128
NCH = 80
EPT = NCH * CH
K = 10
NG = NCH // K

_mesh = plsc.VectorSubcoreMesh(core_axis_name="c", subcore_axis_name="s")
_sc_params = pltpu.CompilerParams(use_tc_tiling_on_sc=False)


def _pack_to_flat(ab, fb):
    @pl.loop(0, 80)
    def _(i):
        for p in range(8):
            fb[i, p * D_HID:(p + 1) * D_HID] = ab[i * 8 + p, :]


def _unpack_from_flat(fb, ab):
    @pl.loop(0, 80)
    def _(i):
        for p in range(8):
            ab[i * 8 + p, :] = fb[i, p * D_HID:(p + 1) * D_HID]


def _deg_body(idx_hbm, out0_hbm, out1_hbm, dst_v, ones_v, zbuf, tbuf, acc,
              sem):
    c = lax.axis_index("c")
    s = lax.axis_index("s")
    wid = s * 2 + c

    @pl.loop(0, 641)
    def _(i):
        zbuf[i, :] = jnp.zeros((16,), jnp.float32)

    @pl.loop(0, CH)
    def _(i):
        ones_v[i, :] = jnp.ones((16,), jnp.float32)

    pltpu.sync_copy(zbuf, acc.at[pl.ds(s * 641, 641)])
    pltpu.sync_copy(idx_hbm.at[1, wid], dst_v)
    plsc.subcore_barrier()

    @pl.loop(0, NCH)
    def _(j):
        pltpu.async_copy(ones_v, acc.at[dst_v.at[j]], sem, add=True)

    @pl.loop(0, NCH)
    def _(j):
        pltpu.make_async_copy(ones_v, acc.at[dst_v.at[0]], sem).wait()

    plsc.subcore_barrier()
    pltpu.sync_copy(acc.at[pl.ds(s * 640, 640)], zbuf.at[pl.ds(0, 640)])
    _pack_to_flat(zbuf, tbuf)

    @pl.when(c == 0)
    def _():
        pltpu.sync_copy(tbuf, out0_hbm.at[pl.ds(s * 80, 80)])

    @pl.when(c == 1)
    def _():
        pltpu.sync_copy(tbuf, out1_hbm.at[pl.ds(s * 80, 80)])


_deg = pl.kernel(
    _deg_body,
    out_type=(jax.ShapeDtypeStruct((NF, 128), jnp.float32),
              jax.ShapeDtypeStruct((NF, 128), jnp.float32)),
    mesh=_mesh,
    scratch_types=[
        pltpu.VMEM((NCH, CH), jnp.int32),
        pltpu.VMEM((CH, D_HID), jnp.float32),
        pltpu.VMEM((641, D_HID), jnp.float32),
        pltpu.VMEM((80, 128), jnp.float32),
        pltpu.VMEM_SHARED((ACC_ROWS, D_HID), jnp.float32),
        pltpu.SemaphoreType.DMA,
    ],
    compiler_params=_sc_params,
)


def _prop_body(gf_hbm, idx_hbm, out0_hbm, out1_hbm, src_v, dst_v, rows, zbuf,
               tbuf, abuf, gsh, acc, gsem0, gsem1, ssem0, ssem1):
    c = lax.axis_index("c")
    s = lax.axis_index("s")
    wid = s * 2 + c

    @pl.loop(0, 641)
    def _(i):
        zbuf[i, :] = jnp.zeros((16,), jnp.float32)

    pltpu.sync_copy(zbuf, acc.at[pl.ds(s * 641, 641)])
    pltpu.sync_copy(gf_hbm.at[pl.ds(s * 80, 80)], tbuf)
    _unpack_from_flat(tbuf, abuf)
    pltpu.sync_copy(abuf, gsh.at[pl.ds(s * 640, 640)])
    pltpu.sync_copy(idx_hbm.at[0, wid], src_v)
    pltpu.sync_copy(idx_hbm.at[1, wid], dst_v)
    plsc.subcore_barrier()

    def fire_gathers(gbase, st, gsem):
        for k in range(K):
            pltpu.async_copy(gsh.at[src_v.at[gbase + k]], rows.at[st, k],
                             gsem)

    def wait_gathers(st, gsem):
        for k in range(K):
            pltpu.make_async_copy(gsh.at[src_v.at[0]], rows.at[st, k],
                                  gsem).wait()

    def fire_scatters(gbase, st, ssem):
        for k in range(K):
            pltpu.async_copy(rows.at[st, k], acc.at[dst_v.at[gbase + k]],
                             ssem, add=True)

    def wait_scatters(st, ssem):
        for k in range(K):
            pltpu.make_async_copy(rows.at[st, k], acc.at[dst_v.at[0]],
                                  ssem).wait()

    fire_gathers(0, 0, gsem0)
    fire_gathers(K, 1, gsem1)
    wait_gathers(0, gsem0)
    fire_scatters(0, 0, ssem0)

    @pl.loop(0, (NG - 2) // 2)
    def _(i):
        ga = 1 + 2 * i
        wait_scatters(0, ssem0)
        fire_gathers((ga + 1) * K, 0, gsem0)
        wait_gathers(1, gsem1)
        fire_scatters(ga * K, 1, ssem1)
        gb = ga + 1
        wait_scatters(1, ssem1)
        fire_gathers((gb + 1) * K, 1, gsem1)
        wait_gathers(0, gsem0)
        fire_scatters(gb * K, 0, ssem0)

    wait_scatters(0, ssem0)
    wait_gathers(1, gsem1)
    fire_scatters((NG - 1) * K, 1, ssem1)
    wait_scatters(1, ssem1)

    plsc.subcore_barrier()
    pltpu.sync_copy(acc.at[pl.ds(s * 640, 640)], abuf)
    _pack_to_flat(abuf, tbuf)

    @pl.when(c == 0)
    def _():
        pltpu.sync_copy(tbuf, out0_hbm.at[pl.ds(s * 80, 80)])

    @pl.when(c == 1)
    def _():
        pltpu.sync_copy(tbuf, out1_hbm.at[pl.ds(s * 80, 80)])


_prop = pl.kernel(
    _prop_body,
    out_type=(jax.ShapeDtypeStruct((NF, 128), jnp.float32),
              jax.ShapeDtypeStruct((NF, 128), jnp.float32)),
    mesh=_mesh,
    scratch_types=[
        pltpu.VMEM((NCH, CH), jnp.int32),
        pltpu.VMEM((NCH, CH), jnp.int32),
        pltpu.VMEM((2, K, CH, D_HID), jnp.float32),
        pltpu.VMEM((641, D_HID), jnp.float32),
        pltpu.VMEM((80, 128), jnp.float32),
        pltpu.VMEM((640, D_HID), jnp.float32),
        pltpu.VMEM_SHARED((NP, D_HID), jnp.float32),
        pltpu.VMEM_SHARED((ACC_ROWS, D_HID), jnp.float32),
        pltpu.SemaphoreType.DMA,
        pltpu.SemaphoreType.DMA,
        pltpu.SemaphoreType.DMA,
        pltpu.SemaphoreType.DMA,
    ],
    compiler_params=_sc_params,
)


def _scale_body(da_ref, db_ref, x_ref, w_ref, g_ref, dis_ref):
    deg = da_ref[...] + db_ref[...] + 1.0
    dis = lax.rsqrt(deg)
    dis_ref[...] = dis
    for k in range(8):
        hk = jnp.dot(x_ref[k], w_ref[...], preferred_element_type=jnp.float32)
        g_ref[:, k * D_HID:(k + 1) * D_HID] = hk
    g_ref[...] = dis * g_ref[...]


def _mid_body(aa_ref, ab_ref, dis_ref, g_ref, b_ref, g2_ref):
    dis = dis_ref[...]
    z = jnp.maximum(dis * (aa_ref[...] + ab_ref[...] + g_ref[...])
                    + b_ref[...], 0.0)
    g2_ref[...] = dis * z


def _out_body(aa_ref, ab_ref, dis_ref, g2_ref, w_ref, b_ref, o_ref):
    a2 = dis_ref[...] * (aa_ref[...] + ab_ref[...] + g2_ref[...])
    for k in range(8):
        v = jnp.dot(a2[:, k * D_HID:(k + 1) * D_HID], w_ref[...],
                    preferred_element_type=jnp.float32) + b_ref[...]
        if (k + 1) * NF <= N:
            o_ref[pl.ds(k * NF, NF)] = v
        else:
            o_ref[pl.ds(k * NF, N - k * NF)] = v[:N - k * NF]


_f32 = jnp.float32
_S = jax.ShapeDtypeStruct

_scale = pl.pallas_call(
    _scale_body,
    out_shape=(_S((NF, 128), _f32), _S((NF, 128), _f32)))
_mid = pl.pallas_call(_mid_body, out_shape=_S((NF, 128), _f32))
_out = pl.pallas_call(_out_body, out_shape=_S((N, D_OUT), _f32))


def kernel(x, edge_index, batch, W1, b1, W2, b2):
    q = jnp.floor((edge_index.astype(jnp.float32) + 0.5)
                  * (1.0 / NF)).astype(jnp.int32)
    ei = (edge_index - q * NF) * 8 + q
    pad_e = EPT * TILES - E
    pad_cols = jnp.concatenate(
        [jnp.zeros((1, pad_e), jnp.int32),
         jnp.full((1, pad_e), JUNK, jnp.int32)])
    idx_r = jnp.concatenate([ei, pad_cols], axis=1).reshape(2, TILES, NCH, CH)
    x_r = jnp.pad(x, ((0, NP - N), (0, 0))).reshape(8, NF, D_IN)
    b1f = jnp.tile(b1, 8).reshape(1, 128)

    dega, degb = _deg(idx_r)
    g1, dis = _scale(dega, degb, x_r, W1)
    a1a, a1b = _prop(g1, idx_r)
    g2 = _mid(a1a, a1b, dis, g1, b1f)
    a2a, a2b = _prop(g2, idx_r)
    return _out(a2a, a2b, dis, g2, W2, b2.reshape(1, D_OUT))
